# Initial kernel scaffold; baseline (speedup 1.0000x reference)
import jax
import jax.numpy as jnp
from jax.experimental import pallas as pl


def kernel(x, edge_index, WX, bX, wA, policy, bn_gamma, bn_beta, Wpred, bpred):
    raise NotImplementedError("write your pallas kernel here")


# speedup vs baseline: 8.7306x; 8.7306x over previous
import functools

import jax
import jax.numpy as jnp
from jax import lax
from jax.experimental import pallas as pl
from jax.experimental.pallas import tpu as pltpu, tpu_sc as plsc

N = 10000
E = 320000
D = 128
D_OUT = 40
EPS = 1e-5

NC = 2
NS = 16
NW = NC * NS
EPW = E // NW
CHUNK = 125
NCH = EPW // CHUNK
NACC = 10240
RPT = NACC // NS

_mesh = plsc.VectorSubcoreMesh(core_axis_name="c", subcore_axis_name="s")


@functools.partial(
    pl.kernel,
    mesh=_mesh,
    out_type=(
        jax.ShapeDtypeStruct((NC * NACC, D), jnp.float32),
        jax.ShapeDtypeStruct((NW * 16,), jnp.int32),
    ),
    scratch_types=[
        pltpu.VMEM((NCH, CHUNK), jnp.int32),
        pltpu.VMEM((NCH, CHUNK), jnp.int32),
        pltpu.VMEM((CHUNK, D), jnp.float32),
        pltpu.VMEM((16,), jnp.int32),
        pltpu.VMEM_SHARED((NACC, D), jnp.float32),
        pltpu.SemaphoreType.DMA,
    ],
)
def _sc_deg(col_hbm, row_hbm, ones_hbm, zeros_hbm, deg_out, rmin_out,
            colv, rowv, vones, rv, dacc, sem):
    c = lax.axis_index("c")
    s = lax.axis_index("s")
    wid = c * NS + s
    pltpu.sync_copy(col_hbm.at[pl.ds(wid * NCH, NCH)], colv)
    pltpu.sync_copy(row_hbm.at[pl.ds(wid * NCH, NCH)], rowv)
    pltpu.sync_copy(ones_hbm, vones)
    pltpu.sync_copy(zeros_hbm, dacc.at[pl.ds(s * RPT, RPT)])
    plsc.subcore_barrier()

    def body(j, m):
        pltpu.sync_copy(vones, dacc.at[colv.at[j]], add=True)
        for k in range(CHUNK // 16):
            m = jnp.minimum(m, rowv[j, pl.ds(k * 16, 16)])
        return m

    m0 = jnp.full((16,), jnp.iinfo(jnp.int32).max, dtype=jnp.int32)
    m = lax.fori_loop(0, NCH, body, m0)
    rv[...] = m
    pltpu.sync_copy(rv, rmin_out.at[pl.ds(wid * 16, 16)])
    plsc.subcore_barrier()
    pltpu.sync_copy(dacc.at[pl.ds(s * RPT, RPT)],
                    deg_out.at[pl.ds(c * NACC + s * RPT, RPT)])


@functools.partial(
    pl.kernel,
    mesh=_mesh,
    out_type=jax.ShapeDtypeStruct((NC * NACC, D), jnp.float32),
    scratch_types=[
        pltpu.VMEM((NCH, CHUNK), jnp.int32),
        pltpu.VMEM((NCH, CHUNK), jnp.int32),
        pltpu.VMEM((CHUNK, D), jnp.float32),
        pltpu.VMEM_SHARED((NACC, D), jnp.float32),
        pltpu.SemaphoreType.DMA,
    ],
)
def _sc_spmm(src_hbm, col_hbm, row_hbm, zeros_hbm, out_hbm,
             colv, rowv, buf, acc, sem):
    c = lax.axis_index("c")
    s = lax.axis_index("s")
    wid = c * NS + s
    pltpu.sync_copy(col_hbm.at[pl.ds(wid * NCH, NCH)], colv)
    pltpu.sync_copy(row_hbm.at[pl.ds(wid * NCH, NCH)], rowv)
    pltpu.sync_copy(zeros_hbm, acc.at[pl.ds(s * RPT, RPT)])
    plsc.subcore_barrier()

    def body(j, _):
        pltpu.async_copy(src_hbm.at[colv.at[j]], buf, sem).wait()
        pltpu.sync_copy(buf, acc.at[rowv.at[j]], add=True)
        return 0

    lax.fori_loop(0, NCH, body, 0)
    plsc.subcore_barrier()
    pltpu.sync_copy(acc.at[pl.ds(s * RPT, RPT)],
                    out_hbm.at[pl.ds(c * NACC + s * RPT, RPT)])


def _spmm(src, col2d, row2d, zeros):
    p = _sc_spmm(src, col2d, row2d, zeros)
    return p[:N] + p[NACC:NACC + N]


ROWS_BLK = 1000
GRID = N // ROWS_BLK


def _linx_body(x_ref, w_ref, b_ref, o_ref):
    o_ref[...] = lax.dot_general(
        x_ref[...], w_ref[...], (((1,), (1,)), ((), ())),
        preferred_element_type=jnp.float32) + b_ref[...]


def _linx(x, WX, bX2):
    return pl.pallas_call(
        _linx_body,
        grid=(GRID,),
        in_specs=[
            pl.BlockSpec((ROWS_BLK, D), lambda i: (i, 0)),
            pl.BlockSpec((D, D), lambda i: (0, 0)),
            pl.BlockSpec((1, D), lambda i: (0, 0)),
        ],
        out_specs=pl.BlockSpec((ROWS_BLK, D), lambda i: (i, 0)),
        out_shape=jax.ShapeDtypeStruct((N, D), jnp.float32),
    )(x, WX, bX2)


def _bn_body(t_ref, g_ref, b_ref, o_ref):
    t = t_ref[...]
    mean = jnp.mean(t, axis=0, keepdims=True)
    var = jnp.mean((t - mean) ** 2, axis=0, keepdims=True)
    o_ref[...] = (t - mean) * lax.rsqrt(var + EPS) * g_ref[...] + b_ref[...]


def _bn(t, g2, b2):
    return pl.pallas_call(
        _bn_body,
        out_shape=jax.ShapeDtypeStruct((N, D), jnp.float32),
    )(t, g2, b2)


def _final_body(h_ref, w_ref, b_ref, o_ref):
    h = jnp.maximum(h_ref[...], 0.0)
    o_ref[...] = lax.dot_general(
        h, w_ref[...], (((1,), (1,)), ((), ())),
        preferred_element_type=jnp.float32) + b_ref[...]


def _final(h, Wpred, bp2):
    return pl.pallas_call(
        _final_body,
        grid=(GRID,),
        in_specs=[
            pl.BlockSpec((ROWS_BLK, D), lambda i: (i, 0)),
            pl.BlockSpec((D_OUT, D), lambda i: (0, 0)),
            pl.BlockSpec((1, D_OUT), lambda i: (0, 0)),
        ],
        out_specs=pl.BlockSpec((ROWS_BLK, D_OUT), lambda i: (i, 0)),
        out_shape=jax.ShapeDtypeStruct((N, D_OUT), jnp.float32),
    )(h, Wpred, bp2)


def kernel(x, edge_index, WX, bX, wA, policy, bn_gamma, bn_beta, Wpred, bpred):
    row = edge_index[0].astype(jnp.int32)
    col = edge_index[1].astype(jnp.int32)
    col2d = col.reshape(E // CHUNK, CHUNK)
    row2d = row.reshape(E // CHUNK, CHUNK)

    onesD = jnp.ones((CHUNK, D), jnp.float32)
    zerosD = jnp.zeros((RPT, D), jnp.float32)

    degp, rminp = _sc_deg(col2d, row2d, onesD, zerosD)
    deg = degp[:N, 0] + degp[NACC:NACC + N, 0]
    rmin = jnp.min(rminp)
    dis = jnp.where(deg > 0, lax.rsqrt(deg), 0.0)
    dis_pad = jnp.concatenate([dis, jnp.zeros_like(dis)])
    dshift = lax.dynamic_slice(dis_pad, (rmin,), (N,))
    row2ds = row2d - rmin

    xX = _linx(x, WX, bX.reshape(1, D))
    pp = jax.nn.softmax(policy[:3])

    y0 = dis[:, None] * xX
    c2 = (dis * dshift)[:, None]
    src = y0
    for _ in range(3):
        src = c2 * _spmm(src, col2d, row2ds, zerosD) + y0
    hX = dshift[:, None] * _spmm(src, col2d, row2ds, zerosD) + xX

    w2 = wA
    hA = jnp.zeros((N, D), jnp.float32)
    for i in range(2):
        t = _spmm(w2, col2d, row2ds, zerosD)
        w2 = _bn(t, bn_gamma[i].reshape(1, D), bn_beta[i].reshape(1, D))
        hA = hA + w2

    hcomb = pp[0] * xX + pp[1] * hX + pp[2] * hA
    return _final(hcomb, Wpred, bpred.reshape(1, D_OUT))


# speedup vs baseline: 12.9621x; 1.4847x over previous
import functools

import jax
import jax.numpy as jnp
from jax import lax
from jax.experimental import pallas as pl
from jax.experimental.pallas import tpu as pltpu, tpu_sc as plsc

N = 10000
E = 320000
D = 128
D_OUT = 40
EPS = 1e-5

NC = 2
NS = 16
NW = NC * NS
EPW = E // NW
CHUNK = 125
NCH = EPW // CHUNK
GS = 8
NG = NCH // GS
NACC = 10240
RPT = NACC // NS

_mesh = plsc.VectorSubcoreMesh(core_axis_name="c", subcore_axis_name="s")


@functools.partial(
    pl.kernel,
    mesh=_mesh,
    out_type=jax.ShapeDtypeStruct((NC * NACC, D), jnp.float32),
    scratch_types=[
        pltpu.VMEM((NG, GS, CHUNK), jnp.int32),
        pltpu.VMEM((CHUNK, D), jnp.float32),
        pltpu.VMEM_SHARED((NACC, D), jnp.float32),
        pltpu.SemaphoreType.DMA,
    ],
)
def _sc_deg(col_hbm, ones_hbm, zeros_hbm, deg_out, colv, vones, dacc, sem):
    c = lax.axis_index("c")
    s = lax.axis_index("s")
    wid = c * NS + s
    pltpu.sync_copy(col_hbm.at[pl.ds(wid * NG, NG)], colv)
    pltpu.sync_copy(ones_hbm, vones)
    pltpu.sync_copy(zeros_hbm, dacc.at[pl.ds(s * RPT, RPT)])
    plsc.subcore_barrier()

    def group(g, _):
        @pl.when(g >= 1)
        def _():
            for b in range(GS):
                pltpu.make_async_copy(
                    vones, dacc.at[colv.at[g - 1, b]], sem).wait()
        for b in range(GS):
            pltpu.async_copy(vones, dacc.at[colv.at[g, b]], sem, add=True)
        return 0

    lax.fori_loop(0, NG, group, 0)
    for b in range(GS):
        pltpu.make_async_copy(vones, dacc.at[colv.at[NG - 1, b]], sem).wait()
    plsc.subcore_barrier()
    pltpu.sync_copy(dacc.at[pl.ds(s * RPT, RPT)],
                    deg_out.at[pl.ds(c * NACC + s * RPT, RPT)])


@functools.partial(
    pl.kernel,
    mesh=_mesh,
    out_type=jax.ShapeDtypeStruct((NC * NACC, D), jnp.float32),
    scratch_types=[
        pltpu.VMEM((GS, CHUNK), jnp.int32),
        pltpu.VMEM((GS, CHUNK), jnp.int32),
        pltpu.VMEM((GS, CHUNK), jnp.int32),
        pltpu.VMEM((GS, CHUNK), jnp.int32),
        pltpu.VMEM((CHUNK, D), jnp.float32),
        pltpu.VMEM((CHUNK, D), jnp.float32),
        pltpu.VMEM_SHARED((NACC, D), jnp.float32),
        pltpu.SemaphoreType.DMA,
        pltpu.SemaphoreType.DMA,
        pltpu.SemaphoreType.DMA,
        pltpu.SemaphoreType.DMA,
        pltpu.SemaphoreType.DMA,
    ],
)
def _sc_spmm(src_hbm, col_hbm, row_hbm, zeros_hbm, out_hbm,
             colA, colB, rowA, rowB, buf0, buf1, acc,
             si, sg0, sg1, ss0, ss1):
    c = lax.axis_index("c")
    s = lax.axis_index("s")
    wid = c * NS + s
    cols = (colA, colB)
    rows = (rowA, rowB)
    bufs = (buf0, buf1)
    sg = (sg0, sg1)
    ss = (ss0, ss1)

    def idx_issue(g, slot):
        pltpu.async_copy(col_hbm.at[wid * NG + g], cols[slot], si)
        pltpu.async_copy(row_hbm.at[wid * NG + g], rows[slot], si)

    def idx_wait(g, slot):
        pltpu.make_async_copy(
            col_hbm.at[wid * NG + g], cols[slot], si).wait()
        pltpu.make_async_copy(
            row_hbm.at[wid * NG + g], rows[slot], si).wait()

    pltpu.sync_copy(zeros_hbm, acc.at[pl.ds(s * RPT, RPT)])
    idx_issue(0, 0)
    idx_wait(0, 0)
    plsc.subcore_barrier()
    pltpu.async_copy(src_hbm.at[cols[0].at[0]], buf0, sg0)

    def pair(gp, _):
        for gg in range(2):
            g = gp * 2 + gg
            for b in range(GS):
                j = g * GS + b
                p = b % 2
                q = 1 - p
                if b == 2:
                    @pl.when(g + 1 < NG)
                    def _():
                        idx_issue(g + 1, 1 - gg)

                @pl.when(j >= 1)
                def _():
                    if b == 0:
                        rprev = rows[1 - gg].at[GS - 1]
                    else:
                        rprev = rows[gg].at[b - 1]
                    pltpu.make_async_copy(bufs[q], acc.at[rprev], ss[q]).wait()

                if b == GS - 1:
                    @pl.when(g + 1 < NG)
                    def _():
                        idx_wait(g + 1, 1 - gg)

                @pl.when(j + 1 < NCH)
                def _():
                    if b == GS - 1:
                        cnext = cols[1 - gg].at[0]
                    else:
                        cnext = cols[gg].at[b + 1]
                    pltpu.async_copy(src_hbm.at[cnext], bufs[q], sg[q])

                pltpu.make_async_copy(
                    src_hbm.at[cols[gg].at[b]], bufs[p], sg[p]).wait()
                pltpu.async_copy(
                    bufs[p], acc.at[rows[gg].at[b]], ss[p], add=True)
        return 0

    lax.fori_loop(0, NG // 2, pair, 0)
    pltpu.make_async_copy(
        bufs[1], acc.at[rows[1].at[GS - 1]], ss[1]).wait()
    plsc.subcore_barrier()
    pltpu.sync_copy(acc.at[pl.ds(s * RPT, RPT)],
                    out_hbm.at[pl.ds(c * NACC + s * RPT, RPT)])


def _spmm(src, col3d, row3d, zeros):
    p = _sc_spmm(src, col3d, row3d, zeros)
    return p[:N] + p[NACC:NACC + N]


ROWS_BLK = 1000
GRID = N // ROWS_BLK


def _linx_body(x_ref, w_ref, b_ref, o_ref):
    o_ref[...] = lax.dot_general(
        x_ref[...], w_ref[...], (((1,), (1,)), ((), ())),
        preferred_element_type=jnp.float32) + b_ref[...]


def _linx(x, WX, bX2):
    return pl.pallas_call(
        _linx_body,
        grid=(GRID,),
        in_specs=[
            pl.BlockSpec((ROWS_BLK, D), lambda i: (i, 0)),
            pl.BlockSpec((D, D), lambda i: (0, 0)),
            pl.BlockSpec((1, D), lambda i: (0, 0)),
        ],
        out_specs=pl.BlockSpec((ROWS_BLK, D), lambda i: (i, 0)),
        out_shape=jax.ShapeDtypeStruct((N, D), jnp.float32),
    )(x, WX, bX2)


def _rmin_body(r_ref, o_ref):
    o_ref[...] = jnp.broadcast_to(jnp.min(r_ref[...]), (8, 128))


def _rmin(row2d):
    return pl.pallas_call(
        _rmin_body,
        out_shape=jax.ShapeDtypeStruct((8, 128), jnp.int32),
    )(row2d)


def _bn_body(t_ref, g_ref, b_ref, o_ref):
    t = t_ref[...]
    mean = jnp.mean(t, axis=0, keepdims=True)
    var = jnp.mean((t - mean) ** 2, axis=0, keepdims=True)
    o_ref[...] = (t - mean) * lax.rsqrt(var + EPS) * g_ref[...] + b_ref[...]


def _bn(t, g2, b2):
    return pl.pallas_call(
        _bn_body,
        out_shape=jax.ShapeDtypeStruct((N, D), jnp.float32),
    )(t, g2, b2)


def _final_body(h_ref, w_ref, b_ref, o_ref):
    h = jnp.maximum(h_ref[...], 0.0)
    o_ref[...] = lax.dot_general(
        h, w_ref[...], (((1,), (1,)), ((), ())),
        preferred_element_type=jnp.float32) + b_ref[...]


def _final(h, Wpred, bp2):
    return pl.pallas_call(
        _final_body,
        grid=(GRID,),
        in_specs=[
            pl.BlockSpec((ROWS_BLK, D), lambda i: (i, 0)),
            pl.BlockSpec((D_OUT, D), lambda i: (0, 0)),
            pl.BlockSpec((1, D_OUT), lambda i: (0, 0)),
        ],
        out_specs=pl.BlockSpec((ROWS_BLK, D_OUT), lambda i: (i, 0)),
        out_shape=jax.ShapeDtypeStruct((N, D_OUT), jnp.float32),
    )(h, Wpred, bp2)


def kernel(x, edge_index, WX, bX, wA, policy, bn_gamma, bn_beta, Wpred, bpred):
    row = edge_index[0].astype(jnp.int32)
    col = edge_index[1].astype(jnp.int32)
    col3d = col.reshape(NW * NG, GS, CHUNK)

    onesD = jnp.ones((CHUNK, D), jnp.float32)
    zerosD = jnp.zeros((RPT, D), jnp.float32)

    degp = _sc_deg(col3d, onesD, zerosD)
    deg = degp[:N, 0] + degp[NACC:NACC + N, 0]
    rmin = _rmin(row.reshape(E // D, D))[0, 0]
    dis = jnp.where(deg > 0, lax.rsqrt(deg), 0.0)
    dis_pad = jnp.concatenate([dis, jnp.zeros_like(dis)])
    dshift = lax.dynamic_slice(dis_pad, (rmin,), (N,))
    row3d = (row - rmin).reshape(NW * NG, GS, CHUNK)

    xX = _linx(x, WX, bX.reshape(1, D))
    pp = jax.nn.softmax(policy[:3])

    y0 = dis[:, None] * xX
    c2 = (dis * dshift)[:, None]
    src = y0
    for _ in range(3):
        src = c2 * _spmm(src, col3d, row3d, zerosD) + y0
    hX = dshift[:, None] * _spmm(src, col3d, row3d, zerosD) + xX

    w2 = wA
    hA = jnp.zeros((N, D), jnp.float32)
    for i in range(2):
        t = _spmm(w2, col3d, row3d, zerosD)
        w2 = _bn(t, bn_gamma[i].reshape(1, D), bn_beta[i].reshape(1, D))
        hA = hA + w2

    hcomb = pp[0] * xX + pp[1] * hX + pp[2] * hA
    return _final(hcomb, Wpred, bpred.reshape(1, D_OUT))


# speedup vs baseline: 13.9570x; 1.0767x over previous
import functools

import jax
import jax.numpy as jnp
from jax import lax
from jax.experimental import pallas as pl
from jax.experimental.pallas import tpu as pltpu, tpu_sc as plsc

N = 10000
E = 320000
D = 128
D_OUT = 40
EPS = 1e-5

NC = 2
NS = 16
NW = NC * NS
EPW = E // NW
CHUNK = 125
NCH = EPW // CHUNK
GS = 8
NG = NCH // GS
NACC = 10240
RPT = NACC // NS

_mesh = plsc.VectorSubcoreMesh(core_axis_name="c", subcore_axis_name="s")


@functools.partial(
    pl.kernel,
    mesh=_mesh,
    out_type=jax.ShapeDtypeStruct((NW * NACC,), jnp.int32),
    scratch_types=[
        pltpu.VMEM((EPW,), jnp.int32),
        pltpu.VMEM((NACC,), jnp.int32),
    ],
    compiler_params=pltpu.CompilerParams(needs_layout_passes=False),
)
def _sc_deg(col_hbm, deg_out, colf, cnt):
    c = lax.axis_index("c")
    s = lax.axis_index("s")
    wid = c * NS + s
    pltpu.sync_copy(col_hbm.at[pl.ds(wid * EPW, EPW)], colf)

    zero16 = jnp.zeros((16,), jnp.int32)

    def zbody(i, _):
        cnt[pl.ds(i * 16, 16)] = zero16
        return 0

    lax.fori_loop(0, NACC // 16, zbody, 0)

    one16 = jnp.ones((16,), jnp.int32)

    def body(i, _):
        idx = colf[pl.ds(i * 16, 16)]
        plsc.addupdate_scatter(cnt, [idx], one16)
        return 0

    lax.fori_loop(0, EPW // 16, body, 0)
    pltpu.sync_copy(cnt, deg_out.at[pl.ds(wid * NACC, NACC)])


@functools.partial(
    pl.kernel,
    mesh=_mesh,
    out_type=jax.ShapeDtypeStruct((NC * NACC, D), jnp.float32),
    scratch_types=[
        pltpu.VMEM((GS, CHUNK), jnp.int32),
        pltpu.VMEM((GS, CHUNK), jnp.int32),
        pltpu.VMEM((GS, CHUNK), jnp.int32),
        pltpu.VMEM((GS, CHUNK), jnp.int32),
        pltpu.VMEM((CHUNK, D), jnp.float32),
        pltpu.VMEM((CHUNK, D), jnp.float32),
        pltpu.VMEM_SHARED((NACC, D), jnp.float32),
        pltpu.SemaphoreType.DMA,
        pltpu.SemaphoreType.DMA,
        pltpu.SemaphoreType.DMA,
        pltpu.SemaphoreType.DMA,
        pltpu.SemaphoreType.DMA,
    ],
)
def _sc_spmm(src_hbm, col_hbm, row_hbm, zeros_hbm, out_hbm,
             colA, colB, rowA, rowB, buf0, buf1, acc,
             si, sg0, sg1, ss0, ss1):
    c = lax.axis_index("c")
    s = lax.axis_index("s")
    wid = c * NS + s
    cols = (colA, colB)
    rows = (rowA, rowB)
    bufs = (buf0, buf1)
    sg = (sg0, sg1)
    ss = (ss0, ss1)

    def idx_issue(g, slot):
        pltpu.async_copy(col_hbm.at[wid * NG + g], cols[slot], si)
        pltpu.async_copy(row_hbm.at[wid * NG + g], rows[slot], si)

    def idx_wait(g, slot):
        pltpu.make_async_copy(
            col_hbm.at[wid * NG + g], cols[slot], si).wait()
        pltpu.make_async_copy(
            row_hbm.at[wid * NG + g], rows[slot], si).wait()

    pltpu.sync_copy(zeros_hbm, acc.at[pl.ds(s * RPT, RPT)])
    idx_issue(0, 0)
    idx_wait(0, 0)
    plsc.subcore_barrier()
    pltpu.async_copy(src_hbm.at[cols[0].at[0]], buf0, sg0)

    def pair(gp, _):
        for gg in range(2):
            g = gp * 2 + gg
            for b in range(GS):
                j = g * GS + b
                p = b % 2
                q = 1 - p
                if b == 2:
                    @pl.when(g + 1 < NG)
                    def _():
                        idx_issue(g + 1, 1 - gg)

                @pl.when(j >= 1)
                def _():
                    if b == 0:
                        rprev = rows[1 - gg].at[GS - 1]
                    else:
                        rprev = rows[gg].at[b - 1]
                    pltpu.make_async_copy(bufs[q], acc.at[rprev], ss[q]).wait()

                if b == GS - 1:
                    @pl.when(g + 1 < NG)
                    def _():
                        idx_wait(g + 1, 1 - gg)

                @pl.when(j + 1 < NCH)
                def _():
                    if b == GS - 1:
                        cnext = cols[1 - gg].at[0]
                    else:
                        cnext = cols[gg].at[b + 1]
                    pltpu.async_copy(src_hbm.at[cnext], bufs[q], sg[q])

                pltpu.make_async_copy(
                    src_hbm.at[cols[gg].at[b]], bufs[p], sg[p]).wait()
                pltpu.async_copy(
                    bufs[p], acc.at[rows[gg].at[b]], ss[p], add=True)
        return 0

    lax.fori_loop(0, NG // 2, pair, 0)
    pltpu.make_async_copy(
        bufs[1], acc.at[rows[1].at[GS - 1]], ss[1]).wait()
    plsc.subcore_barrier()
    pltpu.sync_copy(acc.at[pl.ds(s * RPT, RPT)],
                    out_hbm.at[pl.ds(c * NACC + s * RPT, RPT)])


def _spmm(src, col3d, row3d, zeros):
    p = _sc_spmm(src, col3d, row3d, zeros)
    return p[:N] + p[NACC:NACC + N]


ROWS_BLK = 1000
GRID = N // ROWS_BLK


def _linx_body(x_ref, w_ref, b_ref, o_ref):
    o_ref[...] = lax.dot_general(
        x_ref[...], w_ref[...], (((1,), (1,)), ((), ())),
        preferred_element_type=jnp.float32) + b_ref[...]


def _linx(x, WX, bX2):
    return pl.pallas_call(
        _linx_body,
        grid=(GRID,),
        in_specs=[
            pl.BlockSpec((ROWS_BLK, D), lambda i: (i, 0)),
            pl.BlockSpec((D, D), lambda i: (0, 0)),
            pl.BlockSpec((1, D), lambda i: (0, 0)),
        ],
        out_specs=pl.BlockSpec((ROWS_BLK, D), lambda i: (i, 0)),
        out_shape=jax.ShapeDtypeStruct((N, D), jnp.float32),
    )(x, WX, bX2)


def _rmin_body(r_ref, o_ref):
    o_ref[...] = jnp.broadcast_to(jnp.min(r_ref[...]), (8, 128))


def _rmin(row2d):
    return pl.pallas_call(
        _rmin_body,
        out_shape=jax.ShapeDtypeStruct((8, 128), jnp.int32),
    )(row2d)


def _bn_body(t_ref, g_ref, b_ref, o_ref):
    t = t_ref[...]
    mean = jnp.mean(t, axis=0, keepdims=True)
    var = jnp.mean((t - mean) ** 2, axis=0, keepdims=True)
    o_ref[...] = (t - mean) * lax.rsqrt(var + EPS) * g_ref[...] + b_ref[...]


def _bn(t, g2, b2):
    return pl.pallas_call(
        _bn_body,
        out_shape=jax.ShapeDtypeStruct((N, D), jnp.float32),
    )(t, g2, b2)


def _final_body(h_ref, w_ref, b_ref, o_ref):
    h = jnp.maximum(h_ref[...], 0.0)
    o_ref[...] = lax.dot_general(
        h, w_ref[...], (((1,), (1,)), ((), ())),
        preferred_element_type=jnp.float32) + b_ref[...]


def _final(h, Wpred, bp2):
    return pl.pallas_call(
        _final_body,
        grid=(GRID,),
        in_specs=[
            pl.BlockSpec((ROWS_BLK, D), lambda i: (i, 0)),
            pl.BlockSpec((D_OUT, D), lambda i: (0, 0)),
            pl.BlockSpec((1, D_OUT), lambda i: (0, 0)),
        ],
        out_specs=pl.BlockSpec((ROWS_BLK, D_OUT), lambda i: (i, 0)),
        out_shape=jax.ShapeDtypeStruct((N, D_OUT), jnp.float32),
    )(h, Wpred, bp2)


def kernel(x, edge_index, WX, bX, wA, policy, bn_gamma, bn_beta, Wpred, bpred):
    row = edge_index[0].astype(jnp.int32)
    col = edge_index[1].astype(jnp.int32)
    col3d = col.reshape(NW * NG, GS, CHUNK)

    zerosD = jnp.zeros((RPT, D), jnp.float32)

    degp = _sc_deg(col)
    deg = jnp.sum(degp.reshape(NW, NACC), axis=0)[:N].astype(jnp.float32)
    rmin = _rmin(row.reshape(E // D, D))[0, 0]
    dis = jnp.where(deg > 0, lax.rsqrt(deg), 0.0)
    dis_pad = jnp.concatenate([dis, jnp.zeros_like(dis)])
    dshift = lax.dynamic_slice(dis_pad, (rmin,), (N,))
    row3d = (row - rmin).reshape(NW * NG, GS, CHUNK)

    xX = _linx(x, WX, bX.reshape(1, D))
    pp = jax.nn.softmax(policy[:3])

    y0 = dis[:, None] * xX
    c2 = (dis * dshift)[:, None]
    src = y0
    for _ in range(3):
        src = c2 * _spmm(src, col3d, row3d, zerosD) + y0
    hX = dshift[:, None] * _spmm(src, col3d, row3d, zerosD) + xX

    w2 = wA
    hA = jnp.zeros((N, D), jnp.float32)
    for i in range(2):
        t = _spmm(w2, col3d, row3d, zerosD)
        w2 = _bn(t, bn_gamma[i].reshape(1, D), bn_beta[i].reshape(1, D))
        hA = hA + w2

    hcomb = pp[0] * xX + pp[1] * hX + pp[2] * hA
    return _final(hcomb, Wpred, bpred.reshape(1, D_OUT))
